# baseline (speedup 1.0000x reference)
import functools

import jax
import jax.numpy as jnp
from jax import lax
from jax.experimental import pallas as pl
from jax.experimental.pallas import tpu as pltpu
from jax.experimental.pallas import tpu_sc as plsc

N = 10000
E = 320000
D = 128
HL = 256
NC = 2
NS = 16
LANES = 16
NW = NC * NS
EW = E // NW
CH = 80
NIT = EW // CH
NCHK = N // CH
RB = 5
NG = NIT // RB
RBS = 3
NGS = NIT // RBS
BE = 8000
BN = 2000

_mesh = plsc.VectorSubcoreMesh(
    core_axis_name="c", subcore_axis_name="s", num_cores=NC, num_subcores=NS)


@functools.partial(
    pl.kernel,
    out_type=(
        jax.ShapeDtypeStruct((E, D), jnp.float32),
        jax.ShapeDtypeStruct((E, D), jnp.float32),
    ),
    mesh=_mesh,
    scratch_types=[
        pltpu.VMEM((EW,), jnp.int32),
        pltpu.VMEM((EW,), jnp.int32),
        pltpu.VMEM((2 * RB, CH, D), jnp.float32),
        pltpu.SemaphoreType.DMA,
        pltpu.SemaphoreType.DMA,
    ],
)
def _sc_gather(x_hbm, src_hbm, dst_hbm, xs_out, xd_out,
               si_v, di_v, rows, sem_g, sem_w):
    wid = lax.axis_index("s") * NC + lax.axis_index("c")
    base = wid * EW
    pltpu.sync_copy(src_hbm.at[pl.ds(base, EW)], si_v)
    pltpu.sync_copy(dst_hbm.at[pl.ds(base, EW)], di_v)

    def start_group(g):
        for b in range(RB):
            o = (g * RB + b) * CH
            pltpu.async_copy(x_hbm.at[si_v.at[pl.ds(o, CH)]], rows.at[b],
                             sem_g)
            pltpu.async_copy(x_hbm.at[di_v.at[pl.ds(o, CH)]],
                             rows.at[RB + b], sem_g)

    def drain(sem, k):
        for _ in range(k):
            pltpu.make_async_copy(xs_out.at[pl.ds(0, CH)], rows.at[0],
                                  sem).wait()

    start_group(0)

    @pl.loop(0, NG)
    def _(g):
        drain(sem_g, 2 * RB)
        for b in range(RB):
            off = base + (g * RB + b) * CH
            pltpu.async_copy(rows.at[b], xs_out.at[pl.ds(off, CH)], sem_w)
            pltpu.async_copy(rows.at[RB + b], xd_out.at[pl.ds(off, CH)],
                             sem_w)
        drain(sem_w, 2 * RB)

        @pl.when(g + 1 < NG)
        def _():
            start_group(g + 1)


@functools.partial(
    pl.kernel,
    out_type=jax.ShapeDtypeStruct((NC, N, D), jnp.float32),
    mesh=_mesh,
    scratch_types=[
        pltpu.VMEM((NIT, CH), jnp.int32),
        pltpu.VMEM((RBS, CH, D), jnp.float32),
        pltpu.VMEM_SHARED((N, D), jnp.float32),
        pltpu.SemaphoreType.DMA,
    ],
)
def _sc_scatter(m_hbm, dst_hbm, zd_hbm, agg_out, idx_v, rows, acc_sh, sem_g):
    cid = lax.axis_index("c")
    sid = lax.axis_index("s")
    wid = sid * NC + cid

    @pl.when(sid == 0)
    def _():
        pltpu.sync_copy(zd_hbm, acc_sh)

    pltpu.sync_copy(dst_hbm.at[wid], idx_v)
    plsc.subcore_barrier()
    base = wid * EW

    def load(t, b):
        pltpu.async_copy(m_hbm.at[pl.ds(base + t * CH, CH)], rows.at[b],
                         sem_g)

    def drain(k):
        for _ in range(k):
            pltpu.make_async_copy(m_hbm.at[pl.ds(0, CH)], rows.at[0],
                                  sem_g).wait()

    for b in range(RBS):
        load(b, b)

    @pl.loop(0, NGS)
    def _(g):
        drain(RBS)
        for b in range(RBS):
            t = g * RBS + b
            pltpu.sync_copy(rows.at[b], acc_sh.at[idx_v.at[t]], add=True)

        @pl.when(g + 1 < NGS)
        def _():
            for b in range(RBS):
                load((g + 1) * RBS + b, b)

    for b in range(NIT - NGS * RBS):
        load(NGS * RBS + b, b)
    drain(NIT - NGS * RBS)
    for b in range(NIT - NGS * RBS):
        pltpu.sync_copy(rows.at[b], acc_sh.at[idx_v.at[NGS * RBS + b]],
                        add=True)

    plsc.subcore_barrier()

    @pl.loop(sid, NCHK, step=NS)
    def _(k):
        r0 = k * CH
        pltpu.sync_copy(acc_sh.at[pl.ds(r0, CH)],
                        agg_out.at[cid, pl.ds(r0, CH)])


@functools.partial(
    pl.kernel,
    out_type=jax.ShapeDtypeStruct((NC, N, D), jnp.float32),
    mesh=_mesh,
    scratch_types=[
        pltpu.VMEM((NIT, CH), jnp.int32),
        pltpu.VMEM((CH, D), jnp.float32),
        pltpu.VMEM_SHARED((N, D), jnp.float32),
    ],
)
def _sc_count(dst_hbm, zd_hbm, on_hbm, cnt_out, idx_v, ones_v, cnt_sh):
    cid = lax.axis_index("c")
    sid = lax.axis_index("s")
    wid = sid * NC + cid

    @pl.when(sid == 0)
    def _():
        pltpu.sync_copy(zd_hbm, cnt_sh)

    pltpu.sync_copy(on_hbm, ones_v)
    pltpu.sync_copy(dst_hbm.at[wid], idx_v)
    plsc.subcore_barrier()

    @pl.loop(0, NIT)
    def _(j):
        pltpu.sync_copy(ones_v, cnt_sh.at[idx_v.at[j]], add=True)

    plsc.subcore_barrier()

    @pl.loop(sid, NCHK, step=NS)
    def _(k):
        r0 = k * CH
        pltpu.sync_copy(cnt_sh.at[pl.ds(r0, CH)],
                        cnt_out.at[cid, pl.ds(r0, CH)])


def _bdot(a, b):
    return jnp.dot(a, b, preferred_element_type=jnp.float32)


def _mlp2(xsd, w1, b1, w2, b2):
    h = jnp.maximum(_bdot(xsd, w1) + b1, 0.0)
    return _bdot(h, w2) + b2


_EBLK = pl.BlockSpec((BE, D), lambda i: (i, 0))
_ABLK = pl.BlockSpec((1, 1, BE), lambda i: (i, 0, 0))
_SBLK = pl.BlockSpec((1, 1), lambda i: (0, 0))


def _wspec(shape):
    return pl.BlockSpec(shape, lambda i, _n=len(shape): (0,) * _n)


_W_MLP = [_wspec((2 * D, HL)), _wspec((1, HL)),
          _wspec((HL, D)), _wspec((1, D))]
_EOUT = jax.ShapeDtypeStruct((E, D), jnp.float32)
_SOUT = jax.ShapeDtypeStruct((1, 1), jnp.float32)


def _body_a(xs, xd, w1, b1, w2, b2, m_out):
    xsd = jnp.concatenate([xs[...], xd[...]], axis=1)
    m_out[...] = _mlp2(xsd, w1[...], b1[...], w2[...], b2[...])


_tc_nc = pl.pallas_call(
    _body_a, grid=(E // BE,),
    in_specs=[_EBLK, _EBLK] + _W_MLP,
    out_specs=_EBLK, out_shape=_EOUT)


def _body_b(xs, xd, ang, ew1, eb1, ew2, eb2, wce, wca, bc,
            e_out, s_out):
    xsd = jnp.concatenate([xs[...], xd[...]], axis=1)
    eh = _mlp2(xsd, ew1[...], eb1[...], ew2[...], eb2[...])
    angterm = lax.dot_general(ang[0], wca[...], (((0,), (0,)), ((), ())),
                              preferred_element_type=jnp.float32)
    e = _bdot(eh, wce[...]) + angterm + bc[...]
    e_out[...] = e

    @pl.when(pl.program_id(0) == 0)
    def _():
        s_out[...] = jnp.zeros((1, 1), jnp.float32)

    s_out[...] += jnp.sum(e * e).reshape(1, 1)


_tc_ec1 = pl.pallas_call(
    _body_b, grid=(E // BE,),
    in_specs=[_EBLK, _EBLK, _ABLK] + _W_MLP
             + [_wspec((D, D)), _wspec((1, D)), _wspec((1, D))],
    out_specs=(_EBLK, _SBLK),
    out_shape=(_EOUT, _SOUT))


def _body_c(xs, xd, pe, ew1, eb1, ew2, eb2, wce, wcp, bc,
            e_out, s_out):
    xsd = jnp.concatenate([xs[...], xd[...]], axis=1)
    eh = _mlp2(xsd, ew1[...], eb1[...], ew2[...], eb2[...])
    pev = pe[...]
    e2 = _bdot(eh, wce[...]) + _bdot(pev, wcp[...]) + bc[...]
    e_out[...] = pev + e2

    @pl.when(pl.program_id(0) == 0)
    def _():
        s_out[...] = jnp.zeros((1, 1), jnp.float32)

    s_out[...] += jnp.sum(e2 * e2).reshape(1, 1)


_tc_ec_add = pl.pallas_call(
    _body_c, grid=(E // BE,),
    in_specs=[_EBLK, _EBLK, _EBLK] + _W_MLP
             + [_wspec((D, D)), _wspec((D, D)), _wspec((1, D))],
    out_specs=(_EBLK, _SBLK),
    out_shape=(_EOUT, _SOUT))


def _body_d(xs, xd, pe, ew1, eb1, ew2, eb2, wce, wcp, bc,
            e_out, s_out):
    xsd = jnp.concatenate([xs[...], xd[...]], axis=1)
    eh = _mlp2(xsd, ew1[...], eb1[...], ew2[...], eb2[...])
    e2 = _bdot(eh, wce[...]) + _bdot(pe[...], wcp[...]) + bc[...]
    e_out[...] = e2

    @pl.when(pl.program_id(0) == 0)
    def _():
        s_out[...] = jnp.zeros((1, 1), jnp.float32)

    s_out[...] += jnp.sum(e2 * e2).reshape(1, 1)


_tc_ec3 = pl.pallas_call(
    _body_d, grid=(E // BE,),
    in_specs=[_EBLK, _EBLK, _EBLK] + _W_MLP
             + [_wspec((D, D)), _wspec((D, D)), _wspec((1, D))],
    out_specs=(_EBLK, _SBLK),
    out_shape=(_EOUT, _SOUT))


_NBLK = pl.BlockSpec((BN, D), lambda i: (i, 0))
_AGGB = pl.BlockSpec((NC, BN, D), lambda i: (0, i, 0))
_NOUT = jax.ShapeDtypeStruct((N, D), jnp.float32)


def _body_u1(agg, cnt, x_out, r_out):
    a = agg[0] + agg[1]
    c = cnt[0] + cnt[1]
    rb = 1.0 / jnp.maximum(c, 1.0)
    x_out[...] = a * rb
    r_out[...] = rb


_tc_upd1 = pl.pallas_call(
    _body_u1, grid=(N // BN,),
    in_specs=[_AGGB, _AGGB],
    out_specs=(_NBLK, _NBLK),
    out_shape=(_NOUT, _NOUT))


def _body_u2(agg, r, xp, x_out):
    x_out[...] = xp[...] + (agg[0] + agg[1]) * r[...]


_tc_upd2 = pl.pallas_call(
    _body_u2, grid=(N // BN,),
    in_specs=[_AGGB, _NBLK, _NBLK],
    out_specs=_NBLK, out_shape=_NOUT)


def _mlp_w(layers):
    (w1, b1), (w2, b2) = layers
    return (w1, b1.reshape(1, HL), w2, b2.reshape(1, D))


def kernel(node_features, edge_index, angles, gt_edges, params):
    src = edge_index[0].astype(jnp.int32)
    dst = edge_index[1].astype(jnp.int32)
    dst3 = dst.reshape(NW, NIT, CH)

    nc1 = _mlp_w(params['nc1'])
    nc2 = _mlp_w(params['nc2'])
    nc3 = _mlp_w(params['nc3'])
    ec1 = _mlp_w(params['ec1_m'])
    ec2 = _mlp_w(params['ec2_m'])
    ec3 = _mlp_w(params['ec3_m'])
    (wc1, bc1), = params['ec1_c']
    (wc2, bc2), = params['ec2_c']
    (wc3, bc3), = params['ec3_c']

    zd = jnp.zeros((N, D), jnp.float32)
    on = jnp.ones((CH, D), jnp.float32)

    cnt = _sc_count(dst3, zd, on)
    xs, xd = _sc_gather(node_features, src, dst)
    m1 = _tc_nc(xs, xd, *nc1)
    agg = _sc_scatter(m1, dst3, zd)
    x1, rcnt = _tc_upd1(agg, cnt)

    xs, xd = _sc_gather(x1, src, dst)
    m2 = _tc_nc(xs, xd, *nc2)
    agg = _sc_scatter(m2, dst3, zd)
    e1, s1 = _tc_ec1(xs, xd, angles.reshape(E // BE, 1, BE), *ec1,
                     wc1[:D], wc1[D:], bc1.reshape(1, D))
    x2 = _tc_upd2(agg, rcnt, x1)

    xs, xd = _sc_gather(x2, src, dst)
    m3 = _tc_nc(xs, xd, *nc3)
    agg = _sc_scatter(m3, dst3, zd)
    e12, s2 = _tc_ec_add(xs, xd, e1, *ec2,
                         wc2[:D], wc2[D:], bc2.reshape(1, D))
    x3 = _tc_upd2(agg, rcnt, x2)

    xs, xd = _sc_gather(x3, src, dst)
    e3, s3 = _tc_ec3(xs, xd, e12, *ec3, wc3[:D], wc3[D:], bc3.reshape(1, D))

    denom = jnp.float32(E) * jnp.float32(D)
    side_loss = (s1[0, 0] + s2[0, 0] + s3[0, 0]) / (3.0 * denom)
    return (e3, side_loss)


# speedup vs baseline: 1.0441x; 1.0441x over previous
import functools

import jax
import jax.numpy as jnp
from jax import lax
from jax.experimental import pallas as pl
from jax.experimental.pallas import tpu as pltpu
from jax.experimental.pallas import tpu_sc as plsc

N = 10000
E = 320000
D = 128
HL = 256
NC = 2
NS = 16
LANES = 16
NW = NC * NS
EW = E // NW
CH = 80
NIT = EW // CH
NCHK = N // CH
CHG = 40
NITG = EW // CHG
RB = 5
NG = NITG // RB
RBS = 3
NGS = NIT // RBS
BE = 8000
BN = 2000

_mesh = plsc.VectorSubcoreMesh(
    core_axis_name="c", subcore_axis_name="s", num_cores=NC, num_subcores=NS)


@functools.partial(
    pl.kernel,
    out_type=(
        jax.ShapeDtypeStruct((E, D), jnp.float32),
        jax.ShapeDtypeStruct((E, D), jnp.float32),
    ),
    mesh=_mesh,
    scratch_types=[
        pltpu.VMEM((EW,), jnp.int32),
        pltpu.VMEM((EW,), jnp.int32),
        pltpu.VMEM((4 * RB, CHG, D), jnp.float32),
        pltpu.SemaphoreType.DMA,
        pltpu.SemaphoreType.DMA,
        pltpu.SemaphoreType.DMA,
        pltpu.SemaphoreType.DMA,
    ],
)
def _sc_gather(x_hbm, src_hbm, dst_hbm, xs_out, xd_out,
               si_v, di_v, rows, sem_ga, sem_gb, sem_wa, sem_wb):
    wid = lax.axis_index("s") * NC + lax.axis_index("c")
    base = wid * EW
    pltpu.sync_copy(src_hbm.at[pl.ds(base, EW)], si_v)
    pltpu.sync_copy(dst_hbm.at[pl.ds(base, EW)], di_v)

    def start_reads(g, h, sem):
        for b in range(RB):
            o = (g * RB + b) * CHG
            pltpu.async_copy(x_hbm.at[si_v.at[pl.ds(o, CHG)]],
                             rows.at[2 * RB * h + b], sem)
            pltpu.async_copy(x_hbm.at[di_v.at[pl.ds(o, CHG)]],
                             rows.at[2 * RB * h + RB + b], sem)

    def start_writes(g, h, sem):
        for b in range(RB):
            off = base + (g * RB + b) * CHG
            pltpu.async_copy(rows.at[2 * RB * h + b],
                             xs_out.at[pl.ds(off, CHG)], sem)
            pltpu.async_copy(rows.at[2 * RB * h + RB + b],
                             xd_out.at[pl.ds(off, CHG)], sem)

    def drain(sem, k):
        for _ in range(k):
            pltpu.make_async_copy(xs_out.at[pl.ds(0, CHG)], rows.at[0],
                                  sem).wait()

    start_reads(0, 0, sem_ga)

    @pl.loop(0, NG // 2)
    def _(gg):
        ga = 2 * gg
        gb = 2 * gg + 1

        @pl.when(gg > 0)
        def _():
            drain(sem_wb, 2 * RB)

        start_reads(gb, 1, sem_gb)
        drain(sem_ga, 2 * RB)
        start_writes(ga, 0, sem_wa)
        drain(sem_gb, 2 * RB)
        start_writes(gb, 1, sem_wb)
        drain(sem_wa, 2 * RB)

        @pl.when(gg + 1 < NG // 2)
        def _():
            start_reads(ga + 2, 0, sem_ga)

    drain(sem_wb, 2 * RB)


@functools.partial(
    pl.kernel,
    out_type=jax.ShapeDtypeStruct((NC, N, D), jnp.float32),
    mesh=_mesh,
    scratch_types=[
        pltpu.VMEM((NIT, CH), jnp.int32),
        pltpu.VMEM((RBS, CH, D), jnp.float32),
        pltpu.VMEM_SHARED((N, D), jnp.float32),
        pltpu.SemaphoreType.DMA,
    ],
)
def _sc_scatter(m_hbm, dst_hbm, zd_hbm, agg_out, idx_v, rows, acc_sh, sem_g):
    cid = lax.axis_index("c")
    sid = lax.axis_index("s")
    wid = sid * NC + cid

    @pl.when(sid == 0)
    def _():
        pltpu.sync_copy(zd_hbm, acc_sh)

    pltpu.sync_copy(dst_hbm.at[wid], idx_v)
    plsc.subcore_barrier()
    base = wid * EW

    def load(t, b):
        pltpu.async_copy(m_hbm.at[pl.ds(base + t * CH, CH)], rows.at[b],
                         sem_g)

    def drain(k):
        for _ in range(k):
            pltpu.make_async_copy(m_hbm.at[pl.ds(0, CH)], rows.at[0],
                                  sem_g).wait()

    for b in range(RBS):
        load(b, b)

    @pl.loop(0, NGS)
    def _(g):
        drain(RBS)
        for b in range(RBS):
            t = g * RBS + b
            pltpu.sync_copy(rows.at[b], acc_sh.at[idx_v.at[t]], add=True)

        @pl.when(g + 1 < NGS)
        def _():
            for b in range(RBS):
                load((g + 1) * RBS + b, b)

    for b in range(NIT - NGS * RBS):
        load(NGS * RBS + b, b)
    drain(NIT - NGS * RBS)
    for b in range(NIT - NGS * RBS):
        pltpu.sync_copy(rows.at[b], acc_sh.at[idx_v.at[NGS * RBS + b]],
                        add=True)

    plsc.subcore_barrier()

    @pl.loop(sid, NCHK, step=NS)
    def _(k):
        r0 = k * CH
        pltpu.sync_copy(acc_sh.at[pl.ds(r0, CH)],
                        agg_out.at[cid, pl.ds(r0, CH)])


@functools.partial(
    pl.kernel,
    out_type=jax.ShapeDtypeStruct((NC, N, D), jnp.float32),
    mesh=_mesh,
    scratch_types=[
        pltpu.VMEM((NIT, CH), jnp.int32),
        pltpu.VMEM((CH, D), jnp.float32),
        pltpu.VMEM_SHARED((N, D), jnp.float32),
    ],
)
def _sc_count(dst_hbm, zd_hbm, on_hbm, cnt_out, idx_v, ones_v, cnt_sh):
    cid = lax.axis_index("c")
    sid = lax.axis_index("s")
    wid = sid * NC + cid

    @pl.when(sid == 0)
    def _():
        pltpu.sync_copy(zd_hbm, cnt_sh)

    pltpu.sync_copy(on_hbm, ones_v)
    pltpu.sync_copy(dst_hbm.at[wid], idx_v)
    plsc.subcore_barrier()

    @pl.loop(0, NIT)
    def _(j):
        pltpu.sync_copy(ones_v, cnt_sh.at[idx_v.at[j]], add=True)

    plsc.subcore_barrier()

    @pl.loop(sid, NCHK, step=NS)
    def _(k):
        r0 = k * CH
        pltpu.sync_copy(cnt_sh.at[pl.ds(r0, CH)],
                        cnt_out.at[cid, pl.ds(r0, CH)])


def _bdot(a, b):
    return jnp.dot(a, b, preferred_element_type=jnp.float32)


def _mlp2(xsd, w1, b1, w2, b2):
    h = jnp.maximum(_bdot(xsd, w1) + b1, 0.0)
    return _bdot(h, w2) + b2


_EBLK = pl.BlockSpec((BE, D), lambda i: (i, 0))
_ABLK = pl.BlockSpec((1, 1, BE), lambda i: (i, 0, 0))
_SBLK = pl.BlockSpec((1, 1), lambda i: (0, 0))


def _wspec(shape):
    return pl.BlockSpec(shape, lambda i, _n=len(shape): (0,) * _n)


_W_MLP = [_wspec((2 * D, HL)), _wspec((1, HL)),
          _wspec((HL, D)), _wspec((1, D))]
_EOUT = jax.ShapeDtypeStruct((E, D), jnp.float32)
_SOUT = jax.ShapeDtypeStruct((1, 1), jnp.float32)


def _body_a(xs, xd, w1, b1, w2, b2, m_out):
    xsd = jnp.concatenate([xs[...], xd[...]], axis=1)
    m_out[...] = _mlp2(xsd, w1[...], b1[...], w2[...], b2[...])


_tc_nc = pl.pallas_call(
    _body_a, grid=(E // BE,),
    in_specs=[_EBLK, _EBLK] + _W_MLP,
    out_specs=_EBLK, out_shape=_EOUT)


def _body_b(xs, xd, ang, nw1, nb1, nw2, nb2,
            ew1, eb1, ew2, eb2, wce, wca, bc,
            m_out, e_out, s_out):
    xsd = jnp.concatenate([xs[...], xd[...]], axis=1)
    m_out[...] = _mlp2(xsd, nw1[...], nb1[...], nw2[...], nb2[...])
    eh = _mlp2(xsd, ew1[...], eb1[...], ew2[...], eb2[...])
    angterm = lax.dot_general(ang[0], wca[...], (((0,), (0,)), ((), ())),
                              preferred_element_type=jnp.float32)
    e = _bdot(eh, wce[...]) + angterm + bc[...]
    e_out[...] = e

    @pl.when(pl.program_id(0) == 0)
    def _():
        s_out[...] = jnp.zeros((1, 1), jnp.float32)

    s_out[...] += jnp.sum(e * e).reshape(1, 1)


_tc_nc_ec1 = pl.pallas_call(
    _body_b, grid=(E // BE,),
    in_specs=[_EBLK, _EBLK, _ABLK] + _W_MLP + _W_MLP
             + [_wspec((D, D)), _wspec((1, D)), _wspec((1, D))],
    out_specs=(_EBLK, _EBLK, _SBLK),
    out_shape=(_EOUT, _EOUT, _SOUT))


def _body_c(xs, xd, pe, nw1, nb1, nw2, nb2,
            ew1, eb1, ew2, eb2, wce, wcp, bc,
            m_out, e_out, s_out):
    xsd = jnp.concatenate([xs[...], xd[...]], axis=1)
    m_out[...] = _mlp2(xsd, nw1[...], nb1[...], nw2[...], nb2[...])
    eh = _mlp2(xsd, ew1[...], eb1[...], ew2[...], eb2[...])
    pev = pe[...]
    e2 = _bdot(eh, wce[...]) + _bdot(pev, wcp[...]) + bc[...]
    e_out[...] = pev + e2

    @pl.when(pl.program_id(0) == 0)
    def _():
        s_out[...] = jnp.zeros((1, 1), jnp.float32)

    s_out[...] += jnp.sum(e2 * e2).reshape(1, 1)


_tc_nc_ec2 = pl.pallas_call(
    _body_c, grid=(E // BE,),
    in_specs=[_EBLK, _EBLK, _EBLK] + _W_MLP + _W_MLP
             + [_wspec((D, D)), _wspec((D, D)), _wspec((1, D))],
    out_specs=(_EBLK, _EBLK, _SBLK),
    out_shape=(_EOUT, _EOUT, _SOUT))


def _body_d(xs, xd, pe, ew1, eb1, ew2, eb2, wce, wcp, bc,
            e_out, s_out):
    xsd = jnp.concatenate([xs[...], xd[...]], axis=1)
    eh = _mlp2(xsd, ew1[...], eb1[...], ew2[...], eb2[...])
    e2 = _bdot(eh, wce[...]) + _bdot(pe[...], wcp[...]) + bc[...]
    e_out[...] = e2

    @pl.when(pl.program_id(0) == 0)
    def _():
        s_out[...] = jnp.zeros((1, 1), jnp.float32)

    s_out[...] += jnp.sum(e2 * e2).reshape(1, 1)


_tc_ec3 = pl.pallas_call(
    _body_d, grid=(E // BE,),
    in_specs=[_EBLK, _EBLK, _EBLK] + _W_MLP
             + [_wspec((D, D)), _wspec((D, D)), _wspec((1, D))],
    out_specs=(_EBLK, _SBLK),
    out_shape=(_EOUT, _SOUT))


_NBLK = pl.BlockSpec((BN, D), lambda i: (i, 0))
_AGGB = pl.BlockSpec((NC, BN, D), lambda i: (0, i, 0))
_NOUT = jax.ShapeDtypeStruct((N, D), jnp.float32)


def _body_u1(agg, cnt, x_out, r_out):
    a = agg[0] + agg[1]
    c = cnt[0] + cnt[1]
    rb = 1.0 / jnp.maximum(c, 1.0)
    x_out[...] = a * rb
    r_out[...] = rb


_tc_upd1 = pl.pallas_call(
    _body_u1, grid=(N // BN,),
    in_specs=[_AGGB, _AGGB],
    out_specs=(_NBLK, _NBLK),
    out_shape=(_NOUT, _NOUT))


def _body_u2(agg, r, xp, x_out):
    x_out[...] = xp[...] + (agg[0] + agg[1]) * r[...]


_tc_upd2 = pl.pallas_call(
    _body_u2, grid=(N // BN,),
    in_specs=[_AGGB, _NBLK, _NBLK],
    out_specs=_NBLK, out_shape=_NOUT)


def _mlp_w(layers):
    (w1, b1), (w2, b2) = layers
    return (w1, b1.reshape(1, HL), w2, b2.reshape(1, D))


def kernel(node_features, edge_index, angles, gt_edges, params):
    src = edge_index[0].astype(jnp.int32)
    dst = edge_index[1].astype(jnp.int32)
    dst3 = dst.reshape(NW, NIT, CH)

    nc1 = _mlp_w(params['nc1'])
    nc2 = _mlp_w(params['nc2'])
    nc3 = _mlp_w(params['nc3'])
    ec1 = _mlp_w(params['ec1_m'])
    ec2 = _mlp_w(params['ec2_m'])
    ec3 = _mlp_w(params['ec3_m'])
    (wc1, bc1), = params['ec1_c']
    (wc2, bc2), = params['ec2_c']
    (wc3, bc3), = params['ec3_c']

    zd = jnp.zeros((N, D), jnp.float32)
    on = jnp.ones((CH, D), jnp.float32)

    cnt = _sc_count(dst3, zd, on)
    xs, xd = _sc_gather(node_features, src, dst)
    m1 = _tc_nc(xs, xd, *nc1)
    agg = _sc_scatter(m1, dst3, zd)
    x1, rcnt = _tc_upd1(agg, cnt)

    xs, xd = _sc_gather(x1, src, dst)
    m2, e1, s1 = _tc_nc_ec1(xs, xd, angles.reshape(E // BE, 1, BE), *nc2, *ec1,
                            wc1[:D], wc1[D:], bc1.reshape(1, D))
    agg = _sc_scatter(m2, dst3, zd)
    x2 = _tc_upd2(agg, rcnt, x1)

    xs, xd = _sc_gather(x2, src, dst)
    m3, e12, s2 = _tc_nc_ec2(xs, xd, e1, *nc3, *ec2,
                             wc2[:D], wc2[D:], bc2.reshape(1, D))
    agg = _sc_scatter(m3, dst3, zd)
    x3 = _tc_upd2(agg, rcnt, x2)

    xs, xd = _sc_gather(x3, src, dst)
    e3, s3 = _tc_ec3(xs, xd, e12, *ec3, wc3[:D], wc3[D:], bc3.reshape(1, D))

    denom = jnp.float32(E) * jnp.float32(D)
    side_loss = (s1[0, 0] + s2[0, 0] + s3[0, 0]) / (3.0 * denom)
    return (e3, side_loss)


# speedup vs baseline: 1.0516x; 1.0073x over previous
import functools

import jax
import jax.numpy as jnp
from jax import lax
from jax.experimental import pallas as pl
from jax.experimental.pallas import tpu as pltpu
from jax.experimental.pallas import tpu_sc as plsc

N = 10000
E = 320000
D = 128
HL = 256
NC = 2
NS = 16
LANES = 16
NW = NC * NS
EW = E // NW
CH = 80
NIT = EW // CH
NCHK = N // CH
RB = 5
NG = NIT // RB
RBS = 3
NGS = NIT // RBS
BE = 8000
BN = 2000

_mesh = plsc.VectorSubcoreMesh(
    core_axis_name="c", subcore_axis_name="s", num_cores=NC, num_subcores=NS)


@functools.partial(
    pl.kernel,
    out_type=(
        jax.ShapeDtypeStruct((E, D), jnp.float32),
        jax.ShapeDtypeStruct((E, D), jnp.float32),
    ),
    mesh=_mesh,
    scratch_types=[
        pltpu.VMEM((EW,), jnp.int32),
        pltpu.VMEM((EW,), jnp.int32),
        pltpu.VMEM((2 * RB, CH, D), jnp.float32),
        pltpu.SemaphoreType.DMA,
        pltpu.SemaphoreType.DMA,
    ],
)
def _sc_gather(x_hbm, src_hbm, dst_hbm, xs_out, xd_out,
               si_v, di_v, rows, sem_g, sem_w):
    wid = lax.axis_index("s") * NC + lax.axis_index("c")
    base = wid * EW
    pltpu.sync_copy(src_hbm.at[pl.ds(base, EW)], si_v)
    pltpu.sync_copy(dst_hbm.at[pl.ds(base, EW)], di_v)

    def start_group(g):
        for b in range(RB):
            o = (g * RB + b) * CH
            pltpu.async_copy(x_hbm.at[si_v.at[pl.ds(o, CH)]], rows.at[b],
                             sem_g)
            pltpu.async_copy(x_hbm.at[di_v.at[pl.ds(o, CH)]],
                             rows.at[RB + b], sem_g)

    def drain(sem, k):
        for _ in range(k):
            pltpu.make_async_copy(xs_out.at[pl.ds(0, CH)], rows.at[0],
                                  sem).wait()

    start_group(0)

    @pl.loop(0, NG)
    def _(g):
        drain(sem_g, 2 * RB)
        for b in range(RB):
            off = base + (g * RB + b) * CH
            pltpu.async_copy(rows.at[b], xs_out.at[pl.ds(off, CH)], sem_w)
            pltpu.async_copy(rows.at[RB + b], xd_out.at[pl.ds(off, CH)],
                             sem_w)
        drain(sem_w, 2 * RB)

        @pl.when(g + 1 < NG)
        def _():
            start_group(g + 1)


@functools.partial(
    pl.kernel,
    out_type=jax.ShapeDtypeStruct((NC, N, D), jnp.float32),
    mesh=_mesh,
    scratch_types=[
        pltpu.VMEM((NIT, CH), jnp.int32),
        pltpu.VMEM((RBS, CH, D), jnp.float32),
        pltpu.VMEM_SHARED((N, D), jnp.float32),
        pltpu.SemaphoreType.DMA,
    ],
)
def _sc_scatter(m_hbm, dst_hbm, zd_hbm, agg_out, idx_v, rows, acc_sh, sem_g):
    cid = lax.axis_index("c")
    sid = lax.axis_index("s")
    wid = sid * NC + cid

    @pl.when(sid == 0)
    def _():
        pltpu.sync_copy(zd_hbm, acc_sh)

    pltpu.sync_copy(dst_hbm.at[wid], idx_v)
    plsc.subcore_barrier()
    base = wid * EW

    def load(t, b):
        pltpu.async_copy(m_hbm.at[pl.ds(base + t * CH, CH)], rows.at[b],
                         sem_g)

    def drain(k):
        for _ in range(k):
            pltpu.make_async_copy(m_hbm.at[pl.ds(0, CH)], rows.at[0],
                                  sem_g).wait()

    for b in range(RBS):
        load(b, b)

    @pl.loop(0, NGS)
    def _(g):
        drain(RBS)
        for b in range(RBS):
            t = g * RBS + b
            pltpu.sync_copy(rows.at[b], acc_sh.at[idx_v.at[t]], add=True)

        @pl.when(g + 1 < NGS)
        def _():
            for b in range(RBS):
                load((g + 1) * RBS + b, b)

    for b in range(NIT - NGS * RBS):
        load(NGS * RBS + b, b)
    drain(NIT - NGS * RBS)
    for b in range(NIT - NGS * RBS):
        pltpu.sync_copy(rows.at[b], acc_sh.at[idx_v.at[NGS * RBS + b]],
                        add=True)

    plsc.subcore_barrier()

    @pl.loop(sid, NCHK, step=NS)
    def _(k):
        r0 = k * CH
        pltpu.sync_copy(acc_sh.at[pl.ds(r0, CH)],
                        agg_out.at[cid, pl.ds(r0, CH)])


@functools.partial(
    pl.kernel,
    out_type=jax.ShapeDtypeStruct((NC, N, D), jnp.float32),
    mesh=_mesh,
    scratch_types=[
        pltpu.VMEM((NIT, CH), jnp.int32),
        pltpu.VMEM((CH, D), jnp.float32),
        pltpu.VMEM_SHARED((N, D), jnp.float32),
    ],
)
def _sc_count(dst_hbm, zd_hbm, on_hbm, cnt_out, idx_v, ones_v, cnt_sh):
    cid = lax.axis_index("c")
    sid = lax.axis_index("s")
    wid = sid * NC + cid

    @pl.when(sid == 0)
    def _():
        pltpu.sync_copy(zd_hbm, cnt_sh)

    pltpu.sync_copy(on_hbm, ones_v)
    pltpu.sync_copy(dst_hbm.at[wid], idx_v)
    plsc.subcore_barrier()

    @pl.loop(0, NIT)
    def _(j):
        pltpu.sync_copy(ones_v, cnt_sh.at[idx_v.at[j]], add=True)

    plsc.subcore_barrier()

    @pl.loop(sid, NCHK, step=NS)
    def _(k):
        r0 = k * CH
        pltpu.sync_copy(cnt_sh.at[pl.ds(r0, CH)],
                        cnt_out.at[cid, pl.ds(r0, CH)])


def _bdot(a, b):
    return jnp.dot(a, b, preferred_element_type=jnp.float32)


def _mlp2(xsd, w1, b1, w2, b2):
    h = jnp.maximum(_bdot(xsd, w1) + b1, 0.0)
    return _bdot(h, w2) + b2


_EBLK = pl.BlockSpec((BE, D), lambda i: (i, 0))
_ABLK = pl.BlockSpec((1, 1, BE), lambda i: (i, 0, 0))
_SBLK = pl.BlockSpec((1, 1), lambda i: (0, 0))


def _wspec(shape):
    return pl.BlockSpec(shape, lambda i, _n=len(shape): (0,) * _n)


_W_MLP = [_wspec((2 * D, HL)), _wspec((1, HL)),
          _wspec((HL, D)), _wspec((1, D))]
_EOUT = jax.ShapeDtypeStruct((E, D), jnp.float32)
_SOUT = jax.ShapeDtypeStruct((1, 1), jnp.float32)


def _body_a(xs, xd, w1, b1, w2, b2, m_out):
    xsd = jnp.concatenate([xs[...], xd[...]], axis=1)
    m_out[...] = _mlp2(xsd, w1[...], b1[...], w2[...], b2[...])


_tc_nc = pl.pallas_call(
    _body_a, grid=(E // BE,),
    in_specs=[_EBLK, _EBLK] + _W_MLP,
    out_specs=_EBLK, out_shape=_EOUT)


def _body_b(xs, xd, ang, nw1, nb1, nw2, nb2,
            ew1, eb1, ew2, eb2, wce, wca, bc,
            m_out, e_out, s_out):
    xsd = jnp.concatenate([xs[...], xd[...]], axis=1)
    m_out[...] = _mlp2(xsd, nw1[...], nb1[...], nw2[...], nb2[...])
    eh = _mlp2(xsd, ew1[...], eb1[...], ew2[...], eb2[...])
    angterm = lax.dot_general(ang[0], wca[...], (((0,), (0,)), ((), ())),
                              preferred_element_type=jnp.float32)
    e = _bdot(eh, wce[...]) + angterm + bc[...]
    e_out[...] = e

    @pl.when(pl.program_id(0) == 0)
    def _():
        s_out[...] = jnp.zeros((1, 1), jnp.float32)

    s_out[...] += jnp.sum(e * e).reshape(1, 1)


_tc_nc_ec1 = pl.pallas_call(
    _body_b, grid=(E // BE,),
    in_specs=[_EBLK, _EBLK, _ABLK] + _W_MLP + _W_MLP
             + [_wspec((D, D)), _wspec((1, D)), _wspec((1, D))],
    out_specs=(_EBLK, _EBLK, _SBLK),
    out_shape=(_EOUT, _EOUT, _SOUT))


def _body_c(xs, xd, pe, nw1, nb1, nw2, nb2,
            ew1, eb1, ew2, eb2, wce, wcp, bc,
            m_out, e_out, s_out):
    xsd = jnp.concatenate([xs[...], xd[...]], axis=1)
    m_out[...] = _mlp2(xsd, nw1[...], nb1[...], nw2[...], nb2[...])
    eh = _mlp2(xsd, ew1[...], eb1[...], ew2[...], eb2[...])
    pev = pe[...]
    e2 = _bdot(eh, wce[...]) + _bdot(pev, wcp[...]) + bc[...]
    e_out[...] = pev + e2

    @pl.when(pl.program_id(0) == 0)
    def _():
        s_out[...] = jnp.zeros((1, 1), jnp.float32)

    s_out[...] += jnp.sum(e2 * e2).reshape(1, 1)


_tc_nc_ec2 = pl.pallas_call(
    _body_c, grid=(E // BE,),
    in_specs=[_EBLK, _EBLK, _EBLK] + _W_MLP + _W_MLP
             + [_wspec((D, D)), _wspec((D, D)), _wspec((1, D))],
    out_specs=(_EBLK, _EBLK, _SBLK),
    out_shape=(_EOUT, _EOUT, _SOUT))


def _body_d(xs, xd, pe, ew1, eb1, ew2, eb2, wce, wcp, bc,
            e_out, s_out):
    xsd = jnp.concatenate([xs[...], xd[...]], axis=1)
    eh = _mlp2(xsd, ew1[...], eb1[...], ew2[...], eb2[...])
    e2 = _bdot(eh, wce[...]) + _bdot(pe[...], wcp[...]) + bc[...]
    e_out[...] = e2

    @pl.when(pl.program_id(0) == 0)
    def _():
        s_out[...] = jnp.zeros((1, 1), jnp.float32)

    s_out[...] += jnp.sum(e2 * e2).reshape(1, 1)


_tc_ec3 = pl.pallas_call(
    _body_d, grid=(E // BE,),
    in_specs=[_EBLK, _EBLK, _EBLK] + _W_MLP
             + [_wspec((D, D)), _wspec((D, D)), _wspec((1, D))],
    out_specs=(_EBLK, _SBLK),
    out_shape=(_EOUT, _SOUT))


_NBLK = pl.BlockSpec((BN, D), lambda i: (i, 0))
_AGGB = pl.BlockSpec((NC, BN, D), lambda i: (0, i, 0))
_NOUT = jax.ShapeDtypeStruct((N, D), jnp.float32)


def _body_u1(agg, cnt, x_out, r_out):
    a = agg[0] + agg[1]
    c = cnt[0] + cnt[1]
    rb = 1.0 / jnp.maximum(c, 1.0)
    x_out[...] = a * rb
    r_out[...] = rb


_tc_upd1 = pl.pallas_call(
    _body_u1, grid=(N // BN,),
    in_specs=[_AGGB, _AGGB],
    out_specs=(_NBLK, _NBLK),
    out_shape=(_NOUT, _NOUT))


def _body_u2(agg, r, xp, x_out):
    x_out[...] = xp[...] + (agg[0] + agg[1]) * r[...]


_tc_upd2 = pl.pallas_call(
    _body_u2, grid=(N // BN,),
    in_specs=[_AGGB, _NBLK, _NBLK],
    out_specs=_NBLK, out_shape=_NOUT)


def _mlp_w(layers):
    (w1, b1), (w2, b2) = layers
    return (w1, b1.reshape(1, HL), w2, b2.reshape(1, D))


def kernel(node_features, edge_index, angles, gt_edges, params):
    src = edge_index[0].astype(jnp.int32)
    dst = edge_index[1].astype(jnp.int32)
    dst3 = dst.reshape(NW, NIT, CH)

    nc1 = _mlp_w(params['nc1'])
    nc2 = _mlp_w(params['nc2'])
    nc3 = _mlp_w(params['nc3'])
    ec1 = _mlp_w(params['ec1_m'])
    ec2 = _mlp_w(params['ec2_m'])
    ec3 = _mlp_w(params['ec3_m'])
    (wc1, bc1), = params['ec1_c']
    (wc2, bc2), = params['ec2_c']
    (wc3, bc3), = params['ec3_c']

    zd = jnp.zeros((N, D), jnp.float32)
    on = jnp.ones((CH, D), jnp.float32)

    cnt = _sc_count(dst3, zd, on)
    xs, xd = _sc_gather(node_features, src, dst)
    m1 = _tc_nc(xs, xd, *nc1)
    agg = _sc_scatter(m1, dst3, zd)
    x1, rcnt = _tc_upd1(agg, cnt)

    xs, xd = _sc_gather(x1, src, dst)
    m2, e1, s1 = _tc_nc_ec1(xs, xd, angles.reshape(E // BE, 1, BE), *nc2, *ec1,
                            wc1[:D], wc1[D:], bc1.reshape(1, D))
    agg = _sc_scatter(m2, dst3, zd)
    x2 = _tc_upd2(agg, rcnt, x1)

    xs, xd = _sc_gather(x2, src, dst)
    m3, e12, s2 = _tc_nc_ec2(xs, xd, e1, *nc3, *ec2,
                             wc2[:D], wc2[D:], bc2.reshape(1, D))
    agg = _sc_scatter(m3, dst3, zd)
    x3 = _tc_upd2(agg, rcnt, x2)

    xs, xd = _sc_gather(x3, src, dst)
    e3, s3 = _tc_ec3(xs, xd, e12, *ec3, wc3[:D], wc3[D:], bc3.reshape(1, D))

    denom = jnp.float32(E) * jnp.float32(D)
    side_loss = (s1[0, 0] + s2[0, 0] + s3[0, 0]) / (3.0 * denom)
    return (e3, side_loss)
